# Initial kernel scaffold; baseline (speedup 1.0000x reference)
import jax
import jax.numpy as jnp
from jax.experimental import pallas as pl


def kernel(x, W, b):
    raise NotImplementedError("write your pallas kernel here")


# speedup vs baseline: 6.3829x; 6.3829x over previous
import functools

import jax
import jax.numpy as jnp
from jax import lax
from jax.experimental import pallas as pl
from jax.experimental.pallas import tpu as pltpu
from jax.experimental.pallas import tpu_sc as plsc

_K = 16
_D = 128
_N = 2048
_B = 4
_BLK = 256
_LANES = 16
_CP = 8


def _prep_body(x_ref, w_ref, bb_ref, y_ref, z_ref):
    xb = x_ref[0]
    a = w_ref[:, :_D]
    dm = w_ref[:, _D:] - a
    y_ref[0] = lax.dot_general(
        xb, a, (((0,), (1,)), ((), ())),
        preferred_element_type=jnp.float32,
        precision=lax.Precision.HIGHEST)
    z_ref[0] = lax.dot_general(
        dm, xb, (((1,), (0,)), ((), ())),
        preferred_element_type=jnp.float32,
        precision=lax.Precision.HIGHEST) + bb_ref[...]


def _knn_body(xfull_ref, xblk_ref, idx_ref):
    xb = xfull_ref[0]
    xblk = xblk_ref[0]
    inner = lax.dot_general(
        xblk, xb, (((0,), (0,)), ((), ())),
        preferred_element_type=jnp.float32,
        precision=lax.Precision.DEFAULT)

    xx = jnp.sum(xb * xb, axis=0, keepdims=True)
    score = 2.0 * inner - xx
    cols = lax.broadcasted_iota(jnp.int32, (_BLK, _N), 1)
    base = pl.program_id(0) * _N
    outs = []
    for _ in range(_K):
        m = jnp.max(score, axis=1, keepdims=True)
        cand = jnp.where(score == m, cols, _N)
        am = jnp.min(cand, axis=1, keepdims=True)
        outs.append(am + base)
        score = jnp.where(cols == am, -jnp.inf, score)
    idx_ref[0] = jnp.concatenate(outs, axis=1)


def _finish_body(m_ref, z_ref, o_ref):
    mb = m_ref[0]
    i0 = lax.broadcasted_iota(jnp.int32, (_D, _D), 0)
    i1 = lax.broadcasted_iota(jnp.int32, (_D, _D), 1)
    eye = (i0 == i1).astype(jnp.float32)
    t = lax.dot_general(
        mb, eye, (((0,), (0,)), ((), ())),
        preferred_element_type=jnp.float32,
        precision=lax.Precision.HIGHEST)
    o_ref[0] = jnp.maximum(t + z_ref[0], 0.0)


@functools.lru_cache(maxsize=None)
def _make_gathermax():
    info = plsc.get_sparse_core_info()
    nc = info.num_cores
    nw = nc * info.num_subcores
    p = _B * _N
    ppw = p // nw
    nch = ppw // _CP
    mesh = plsc.VectorSubcoreMesh(core_axis_name="c", subcore_axis_name="s")

    @functools.partial(
        pl.kernel,
        mesh=mesh,
        out_type=jax.ShapeDtypeStruct((p, _D), jnp.float32),
        scratch_types=[
            pltpu.VMEM((_CP * _K,), jnp.int32),
            pltpu.VMEM((_CP * _K, _D), jnp.float32),
            pltpu.VMEM((_CP, _D), jnp.float32),
            pltpu.SemaphoreType.DMA,
        ],
    )
    def gathermax(y_hbm, gidx_hbm, m_hbm, idx_v, rows_v, out_v, sem):
        wid = lax.axis_index("s") * nc + lax.axis_index("c")
        base = wid * ppw

        def chunk(g, carry):
            off = (base + g * _CP) * _K
            pltpu.sync_copy(gidx_hbm.at[pl.ds(off, _CP * _K)], idx_v)
            pltpu.async_copy(y_hbm.at[idx_v], rows_v, sem).wait()
            for pt in range(_CP):
                for c in range(_D // _LANES):
                    sl = pl.ds(c * _LANES, _LANES)
                    vals = [rows_v[pt * _K + r, sl] for r in range(_K)]
                    while len(vals) > 1:
                        vals = [jnp.maximum(vals[i], vals[i + 1])
                                if i + 1 < len(vals) else vals[i]
                                for i in range(0, len(vals), 2)]
                    out_v[pt, sl] = vals[0]
            pltpu.sync_copy(out_v, m_hbm.at[pl.ds(base + g * _CP, _CP)])
            return carry

        lax.fori_loop(0, nch, chunk, 0)

    return gathermax


def kernel(x, W, b):
    bb = b.reshape(_D, 1)
    y, z = pl.pallas_call(
        _prep_body,
        grid=(_B,),
        in_specs=[
            pl.BlockSpec((1, _D, _N), lambda i: (i, 0, 0)),
            pl.BlockSpec((_D, 2 * _D), lambda i: (0, 0)),
            pl.BlockSpec((_D, 1), lambda i: (0, 0)),
        ],
        out_specs=[
            pl.BlockSpec((1, _N, _D), lambda i: (i, 0, 0)),
            pl.BlockSpec((1, _D, _N), lambda i: (i, 0, 0)),
        ],
        out_shape=[
            jax.ShapeDtypeStruct((_B, _N, _D), jnp.float32),
            jax.ShapeDtypeStruct((_B, _D, _N), jnp.float32),
        ],
    )(x, W, bb)

    idx = pl.pallas_call(
        _knn_body,
        grid=(_B, _N // _BLK),
        in_specs=[
            pl.BlockSpec((1, _D, _N), lambda bq, i: (bq, 0, 0)),
            pl.BlockSpec((1, _D, _BLK), lambda bq, i: (bq, 0, i)),
        ],
        out_specs=pl.BlockSpec((1, _BLK, _K), lambda bq, i: (bq, i, 0)),
        out_shape=jax.ShapeDtypeStruct((_B, _N, _K), jnp.int32),
    )(x, x)

    m = _make_gathermax()(y.reshape(_B * _N, _D),
                          idx.reshape(_B * _N * _K))

    out = pl.pallas_call(
        _finish_body,
        grid=(_B, _N // _D),
        in_specs=[
            pl.BlockSpec((1, _D, _D), lambda bq, i: (bq, i, 0)),
            pl.BlockSpec((1, _D, _D), lambda bq, i: (bq, 0, i)),
        ],
        out_specs=pl.BlockSpec((1, _D, _D), lambda bq, i: (bq, 0, i)),
        out_shape=jax.ShapeDtypeStruct((_B, _D, _N), jnp.float32),
    )(m.reshape(_B, _N, _D), z)
    return out


# speedup vs baseline: 7.2486x; 1.1356x over previous
import functools

import jax
import jax.numpy as jnp
from jax import lax
from jax.experimental import pallas as pl
from jax.experimental.pallas import tpu as pltpu
from jax.experimental.pallas import tpu_sc as plsc

_K = 16
_D = 128
_N = 2048
_B = 4
_BLK = 256
_LANES = 16
_CP = 8


def _prep_body(x_ref, w_ref, bb_ref, y_ref, z_ref):
    xb = x_ref[0]
    a = w_ref[:, :_D]
    dm = w_ref[:, _D:] - a
    y_ref[0] = lax.dot_general(
        xb, a, (((0,), (1,)), ((), ())),
        preferred_element_type=jnp.float32,
        precision=lax.Precision.HIGHEST)
    z_ref[0] = lax.dot_general(
        dm, xb, (((1,), (0,)), ((), ())),
        preferred_element_type=jnp.float32,
        precision=lax.Precision.HIGHEST) + bb_ref[...]


def _knn_body(xfull_ref, xblk_ref, idx_ref):
    xb = xfull_ref[0]
    xblk = xblk_ref[0]
    inner = lax.dot_general(
        xblk, xb, (((0,), (0,)), ((), ())),
        preferred_element_type=jnp.float32,
        precision=lax.Precision.DEFAULT)

    xx = jnp.sum(xb * xb, axis=0, keepdims=True)
    score = 2.0 * inner - xx
    cols = lax.broadcasted_iota(jnp.int32, (_BLK, _N), 1)
    base = pl.program_id(0) * _N
    outs = []
    for _ in range(_K):
        m = jnp.max(score, axis=1, keepdims=True)
        cand = jnp.where(score == m, cols, _N)
        am = jnp.min(cand, axis=1, keepdims=True)
        outs.append(am + base)
        score = jnp.where(cols == am, -jnp.inf, score)
    idx_ref[0] = jnp.concatenate(outs, axis=1)


def _finish_body(m_ref, z_ref, o_ref):
    mb = m_ref[0]
    i0 = lax.broadcasted_iota(jnp.int32, (_D, _D), 0)
    i1 = lax.broadcasted_iota(jnp.int32, (_D, _D), 1)
    eye = (i0 == i1).astype(jnp.float32)
    t = lax.dot_general(
        mb, eye, (((0,), (0,)), ((), ())),
        preferred_element_type=jnp.float32,
        precision=lax.Precision.HIGHEST)
    o_ref[0] = jnp.maximum(t + z_ref[0], 0.0)


@functools.lru_cache(maxsize=None)
def _make_gathermax():
    info = plsc.get_sparse_core_info()
    nc = info.num_cores
    nw = nc * info.num_subcores
    p = _B * _N
    ppw = p // nw
    nch = ppw // _CP
    mesh = plsc.VectorSubcoreMesh(core_axis_name="c", subcore_axis_name="s")

    @functools.partial(
        pl.kernel,
        mesh=mesh,
        out_type=jax.ShapeDtypeStruct((p, _D), jnp.float32),
        scratch_types=[
            pltpu.VMEM((2, _CP * _K), jnp.int32),
            pltpu.VMEM((2, _CP * _K, _D), jnp.float32),
            pltpu.VMEM((ppw, _D), jnp.float32),
            pltpu.SemaphoreType.DMA,
            pltpu.SemaphoreType.DMA,
        ],
    )
    def gathermax(y_hbm, gidx_hbm, m_hbm, idx_v, rows_v, out_v, sem0, sem1):
        sems = (sem0, sem1)
        wid = lax.axis_index("s") * nc + lax.axis_index("c")
        base = wid * ppw

        def issue(g, slot):
            off = (base + g * _CP) * _K
            pltpu.sync_copy(gidx_hbm.at[pl.ds(off, _CP * _K)], idx_v.at[slot])
            pltpu.make_async_copy(
                y_hbm.at[idx_v.at[slot]], rows_v.at[slot], sems[slot]).start()

        for slot in range(2):
            issue(slot, slot)

        def outer(i, carry):
            for slot in range(2):
                g = i * 2 + slot
                pltpu.make_async_copy(
                    y_hbm.at[idx_v.at[slot]], rows_v.at[slot],
                    sems[slot]).wait()
                for pt in range(_CP):
                    for c in range(_D // _LANES):
                        sl = pl.ds(c * _LANES, _LANES)
                        vals = [rows_v[slot, pt * _K + r, sl]
                                for r in range(_K)]
                        while len(vals) > 1:
                            vals = [jnp.maximum(vals[i2], vals[i2 + 1])
                                    if i2 + 1 < len(vals) else vals[i2]
                                    for i2 in range(0, len(vals), 2)]
                        out_v[g * _CP + pt, sl] = vals[0]
                gn = g + 2
                @pl.when(gn < nch)
                def _():
                    issue(gn, slot)
            return carry

        lax.fori_loop(0, nch // 2, outer, 0)
        pltpu.sync_copy(out_v, m_hbm.at[pl.ds(base, ppw)])

    return gathermax


def kernel(x, W, b):
    bb = b.reshape(_D, 1)
    y, z = pl.pallas_call(
        _prep_body,
        grid=(_B,),
        in_specs=[
            pl.BlockSpec((1, _D, _N), lambda i: (i, 0, 0)),
            pl.BlockSpec((_D, 2 * _D), lambda i: (0, 0)),
            pl.BlockSpec((_D, 1), lambda i: (0, 0)),
        ],
        out_specs=[
            pl.BlockSpec((1, _N, _D), lambda i: (i, 0, 0)),
            pl.BlockSpec((1, _D, _N), lambda i: (i, 0, 0)),
        ],
        out_shape=[
            jax.ShapeDtypeStruct((_B, _N, _D), jnp.float32),
            jax.ShapeDtypeStruct((_B, _D, _N), jnp.float32),
        ],
    )(x, W, bb)

    idx = pl.pallas_call(
        _knn_body,
        grid=(_B, _N // _BLK),
        in_specs=[
            pl.BlockSpec((1, _D, _N), lambda bq, i: (bq, 0, 0)),
            pl.BlockSpec((1, _D, _BLK), lambda bq, i: (bq, 0, i)),
        ],
        out_specs=pl.BlockSpec((1, _BLK, _K), lambda bq, i: (bq, i, 0)),
        out_shape=jax.ShapeDtypeStruct((_B, _N, _K), jnp.int32),
    )(x, x)

    m = _make_gathermax()(y.reshape(_B * _N, _D),
                          idx.reshape(_B * _N * _K))

    out = pl.pallas_call(
        _finish_body,
        grid=(_B, _N // _D),
        in_specs=[
            pl.BlockSpec((1, _D, _D), lambda bq, i: (bq, i, 0)),
            pl.BlockSpec((1, _D, _D), lambda bq, i: (bq, 0, i)),
        ],
        out_specs=pl.BlockSpec((1, _D, _D), lambda bq, i: (bq, 0, i)),
        out_shape=jax.ShapeDtypeStruct((_B, _D, _N), jnp.float32),
    )(m.reshape(_B, _N, _D), z)
    return out


# speedup vs baseline: 8.4781x; 1.1696x over previous
import functools

import jax
import jax.numpy as jnp
from jax import lax
from jax.experimental import pallas as pl
from jax.experimental.pallas import tpu as pltpu
from jax.experimental.pallas import tpu_sc as plsc

_K = 16
_D = 128
_N = 2048
_B = 4
_BLK = 256
_LANES = 16
_CP = 8


def _prep_body(x_ref, w_ref, bb_ref, y_ref, z_ref):
    xb = x_ref[0]
    a = w_ref[:, :_D]
    dm = w_ref[:, _D:] - a
    y_ref[0] = lax.dot_general(
        xb, a, (((0,), (1,)), ((), ())),
        preferred_element_type=jnp.float32,
        precision=lax.Precision.HIGHEST)
    z_ref[0] = lax.dot_general(
        dm, xb, (((1,), (0,)), ((), ())),
        preferred_element_type=jnp.float32,
        precision=lax.Precision.HIGHEST) + bb_ref[...]


def _knn_body(xfull_ref, xblk_ref, idx_ref):
    xb = xfull_ref[0]
    xblk = xblk_ref[0]
    inner = lax.dot_general(
        xblk, xb, (((0,), (0,)), ((), ())),
        preferred_element_type=jnp.float32,
        precision=lax.Precision.DEFAULT)

    xx = jnp.sum(xb * xb, axis=0, keepdims=True)
    score = 2.0 * inner - xx
    colsf = lax.broadcasted_iota(jnp.int32, (_BLK, _N), 1).astype(jnp.float32)
    base = pl.program_id(0) * _N
    big = jnp.float32(float(_N))
    outs = []
    for _ in range(_K):
        m = jnp.max(score, axis=1, keepdims=True)
        cand = jnp.where(score == m, colsf, big)
        amf = jnp.min(cand, axis=1, keepdims=True)
        outs.append(amf)
        score = jnp.where(cand == amf, -jnp.inf, score)
    idxf = jnp.concatenate(outs, axis=1)
    idx_ref[0] = idxf.astype(jnp.int32) + base


def _finish_body(m_ref, z_ref, o_ref):
    mb = m_ref[0]
    i0 = lax.broadcasted_iota(jnp.int32, (_D, _D), 0)
    i1 = lax.broadcasted_iota(jnp.int32, (_D, _D), 1)
    eye = (i0 == i1).astype(jnp.float32)
    t = lax.dot_general(
        mb, eye, (((0,), (0,)), ((), ())),
        preferred_element_type=jnp.float32,
        precision=lax.Precision.HIGHEST)
    o_ref[0] = jnp.maximum(t + z_ref[0], 0.0)


@functools.lru_cache(maxsize=None)
def _make_gathermax():
    info = plsc.get_sparse_core_info()
    nc = info.num_cores
    nw = nc * info.num_subcores
    p = _B * _N
    ppw = p // nw
    nch = ppw // _CP
    mesh = plsc.VectorSubcoreMesh(core_axis_name="c", subcore_axis_name="s")

    @functools.partial(
        pl.kernel,
        mesh=mesh,
        out_type=jax.ShapeDtypeStruct((p, _D), jnp.float32),
        scratch_types=[
            pltpu.VMEM((2, _CP * _K), jnp.int32),
            pltpu.VMEM((2, _CP * _K, _D), jnp.float32),
            pltpu.VMEM((ppw, _D), jnp.float32),
            pltpu.SemaphoreType.DMA,
            pltpu.SemaphoreType.DMA,
        ],
    )
    def gathermax(y_hbm, gidx_hbm, m_hbm, idx_v, rows_v, out_v, sem0, sem1):
        sems = (sem0, sem1)
        wid = lax.axis_index("s") * nc + lax.axis_index("c")
        base = wid * ppw

        def issue(g, slot):
            off = (base + g * _CP) * _K
            pltpu.sync_copy(gidx_hbm.at[pl.ds(off, _CP * _K)], idx_v.at[slot])
            pltpu.make_async_copy(
                y_hbm.at[idx_v.at[slot]], rows_v.at[slot], sems[slot]).start()

        for slot in range(2):
            issue(slot, slot)

        def outer(i, carry):
            for slot in range(2):
                g = i * 2 + slot
                pltpu.make_async_copy(
                    y_hbm.at[idx_v.at[slot]], rows_v.at[slot],
                    sems[slot]).wait()
                for pt in range(_CP):
                    for c in range(_D // _LANES):
                        sl = pl.ds(c * _LANES, _LANES)
                        vals = [rows_v[slot, pt * _K + r, sl]
                                for r in range(_K)]
                        while len(vals) > 1:
                            vals = [jnp.maximum(vals[i2], vals[i2 + 1])
                                    if i2 + 1 < len(vals) else vals[i2]
                                    for i2 in range(0, len(vals), 2)]
                        out_v[g * _CP + pt, sl] = vals[0]
                gn = g + 2
                @pl.when(gn < nch)
                def _():
                    issue(gn, slot)
            return carry

        lax.fori_loop(0, nch // 2, outer, 0)
        pltpu.sync_copy(out_v, m_hbm.at[pl.ds(base, ppw)])

    return gathermax


def kernel(x, W, b):
    bb = b.reshape(_D, 1)
    y, z = pl.pallas_call(
        _prep_body,
        grid=(_B,),
        in_specs=[
            pl.BlockSpec((1, _D, _N), lambda i: (i, 0, 0)),
            pl.BlockSpec((_D, 2 * _D), lambda i: (0, 0)),
            pl.BlockSpec((_D, 1), lambda i: (0, 0)),
        ],
        out_specs=[
            pl.BlockSpec((1, _N, _D), lambda i: (i, 0, 0)),
            pl.BlockSpec((1, _D, _N), lambda i: (i, 0, 0)),
        ],
        out_shape=[
            jax.ShapeDtypeStruct((_B, _N, _D), jnp.float32),
            jax.ShapeDtypeStruct((_B, _D, _N), jnp.float32),
        ],
    )(x, W, bb)

    idx = pl.pallas_call(
        _knn_body,
        grid=(_B, _N // _BLK),
        in_specs=[
            pl.BlockSpec((1, _D, _N), lambda bq, i: (bq, 0, 0)),
            pl.BlockSpec((1, _D, _BLK), lambda bq, i: (bq, 0, i)),
        ],
        out_specs=pl.BlockSpec((1, _BLK, _K), lambda bq, i: (bq, i, 0)),
        out_shape=jax.ShapeDtypeStruct((_B, _N, _K), jnp.int32),
    )(x, x)

    m = _make_gathermax()(y.reshape(_B * _N, _D),
                          idx.reshape(_B * _N * _K))

    out = pl.pallas_call(
        _finish_body,
        grid=(_B, _N // _D),
        in_specs=[
            pl.BlockSpec((1, _D, _D), lambda bq, i: (bq, i, 0)),
            pl.BlockSpec((1, _D, _D), lambda bq, i: (bq, 0, i)),
        ],
        out_specs=pl.BlockSpec((1, _D, _D), lambda bq, i: (bq, 0, i)),
        out_shape=jax.ShapeDtypeStruct((_B, _D, _N), jnp.float32),
    )(m.reshape(_B, _N, _D), z)
    return out


# speedup vs baseline: 10.4114x; 1.2280x over previous
import functools

import jax
import jax.numpy as jnp
from jax import lax
from jax.experimental import pallas as pl
from jax.experimental.pallas import tpu as pltpu
from jax.experimental.pallas import tpu_sc as plsc

_K = 16
_D = 128
_N = 2048
_B = 4
_BLK = 256
_LANES = 16
_CP = 8


def _prep_body(x_ref, w_ref, bb_ref, y_ref, z_ref):
    xb = x_ref[0]
    a = w_ref[:, :_D]
    dm = w_ref[:, _D:] - a
    y_ref[0] = lax.dot_general(
        xb, a, (((0,), (1,)), ((), ())),
        preferred_element_type=jnp.float32,
        precision=lax.Precision.HIGHEST)
    z_ref[0] = lax.dot_general(
        dm, xb, (((1,), (0,)), ((), ())),
        preferred_element_type=jnp.float32,
        precision=lax.Precision.HIGHEST) + bb_ref[...]


def _knn_body(xfull_ref, xblk_ref, idx_ref, *, base):
    xb = xfull_ref[0]
    xblk = xblk_ref[0]
    inner = lax.dot_general(
        xblk, xb, (((0,), (0,)), ((), ())),
        preferred_element_type=jnp.float32,
        precision=lax.Precision.DEFAULT)

    xx = jnp.sum(xb * xb, axis=0, keepdims=True)
    score = 2.0 * inner - xx
    colsf = lax.broadcasted_iota(jnp.int32, (_BLK, _N), 1).astype(jnp.float32)
    big = jnp.float32(float(_N))
    outs = []
    for _ in range(_K):
        m = jnp.max(score, axis=1, keepdims=True)
        cand = jnp.where(score == m, colsf, big)
        amf = jnp.min(cand, axis=1, keepdims=True)
        outs.append(amf)
        score = jnp.where(cand == amf, -jnp.inf, score)
    idxf = jnp.concatenate(outs, axis=1)
    idx_ref[...] = idxf.astype(jnp.int32) + base


def _finish_body(m_ref, z_ref, o_ref):
    mb = m_ref[...]
    i0 = lax.broadcasted_iota(jnp.int32, (_D, _D), 0)
    i1 = lax.broadcasted_iota(jnp.int32, (_D, _D), 1)
    eye = (i0 == i1).astype(jnp.float32)
    t = lax.dot_general(
        mb, eye, (((0,), (0,)), ((), ())),
        preferred_element_type=jnp.float32,
        precision=lax.Precision.HIGHEST)
    o_ref[...] = jnp.maximum(t + z_ref[0], 0.0)


@functools.lru_cache(maxsize=None)
def _make_gathermax(p):
    info = plsc.get_sparse_core_info()
    nc = info.num_cores
    nw = nc * info.num_subcores
    ppw = p // nw
    nch = ppw // _CP
    mesh = plsc.VectorSubcoreMesh(core_axis_name="c", subcore_axis_name="s")

    @functools.partial(
        pl.kernel,
        mesh=mesh,
        out_type=jax.ShapeDtypeStruct((p, _D), jnp.float32),
        scratch_types=[
            pltpu.VMEM((2, _CP * _K), jnp.int32),
            pltpu.VMEM((2, _CP * _K, _D), jnp.float32),
            pltpu.VMEM((ppw, _D), jnp.float32),
            pltpu.SemaphoreType.DMA,
            pltpu.SemaphoreType.DMA,
        ],
    )
    def gathermax(y_hbm, gidx_hbm, m_hbm, idx_v, rows_v, out_v, sem0, sem1):
        sems = (sem0, sem1)
        wid = lax.axis_index("s") * nc + lax.axis_index("c")
        base = wid * ppw

        def issue(g, slot):
            off = (base + g * _CP) * _K
            pltpu.sync_copy(gidx_hbm.at[pl.ds(off, _CP * _K)], idx_v.at[slot])
            pltpu.make_async_copy(
                y_hbm.at[idx_v.at[slot]], rows_v.at[slot], sems[slot]).start()

        for slot in range(2):
            issue(slot, slot)

        def outer(i, carry):
            for slot in range(2):
                g = i * 2 + slot
                pltpu.make_async_copy(
                    y_hbm.at[idx_v.at[slot]], rows_v.at[slot],
                    sems[slot]).wait()
                for pt in range(_CP):
                    for c in range(_D // _LANES):
                        sl = pl.ds(c * _LANES, _LANES)
                        vals = [rows_v[slot, pt * _K + r, sl]
                                for r in range(_K)]
                        while len(vals) > 1:
                            vals = [jnp.maximum(vals[i2], vals[i2 + 1])
                                    if i2 + 1 < len(vals) else vals[i2]
                                    for i2 in range(0, len(vals), 2)]
                        out_v[g * _CP + pt, sl] = vals[0]
                gn = g + 2
                @pl.when(gn < nch)
                def _():
                    issue(gn, slot)
            return carry

        lax.fori_loop(0, nch // 2, outer, 0)
        pltpu.sync_copy(out_v, m_hbm.at[pl.ds(base, ppw)])

    return gathermax


def kernel(x, W, b):
    bb = b.reshape(_D, 1)
    y, z = pl.pallas_call(
        _prep_body,
        grid=(_B,),
        in_specs=[
            pl.BlockSpec((1, _D, _N), lambda i: (i, 0, 0)),
            pl.BlockSpec((_D, 2 * _D), lambda i: (0, 0)),
            pl.BlockSpec((_D, 1), lambda i: (0, 0)),
        ],
        out_specs=[
            pl.BlockSpec((1, _N, _D), lambda i: (i, 0, 0)),
            pl.BlockSpec((1, _D, _N), lambda i: (i, 0, 0)),
        ],
        out_shape=[
            jax.ShapeDtypeStruct((_B, _N, _D), jnp.float32),
            jax.ShapeDtypeStruct((_B, _D, _N), jnp.float32),
        ],
    )(x, W, bb)

    y_flat = y.reshape(_B * _N, _D)
    gm = _make_gathermax(_N)
    outs = []
    for bq in range(_B):
        idx_b = pl.pallas_call(
            functools.partial(_knn_body, base=bq * _N),
            grid=(_N // _BLK,),
            in_specs=[
                pl.BlockSpec((1, _D, _N), lambda i, bq=bq: (bq, 0, 0)),
                pl.BlockSpec((1, _D, _BLK), lambda i, bq=bq: (bq, 0, i)),
            ],
            out_specs=pl.BlockSpec((_BLK, _K), lambda i: (i, 0)),
            out_shape=jax.ShapeDtypeStruct((_N, _K), jnp.int32),
        )(x, x)
        m_b = gm(y_flat, idx_b.reshape(_N * _K))
        out_b = pl.pallas_call(
            _finish_body,
            grid=(_N // _D,),
            in_specs=[
                pl.BlockSpec((_D, _D), lambda i: (i, 0)),
                pl.BlockSpec((1, _D, _D), lambda i, bq=bq: (bq, 0, i)),
            ],
            out_specs=pl.BlockSpec((_D, _D), lambda i: (0, i)),
            out_shape=jax.ShapeDtypeStruct((_D, _N), jnp.float32),
        )(m_b, z)
        outs.append(out_b)
    return jnp.stack(outs)
